# Initial kernel scaffold; baseline (speedup 1.0000x reference)
import jax
import jax.numpy as jnp
from jax.experimental import pallas as pl


def kernel(x, edge_index, Wl0, Wr0, att0, b0, g0, be0, Wl1, Wr1, att1, b1, g1, be1, Wl2, Wr2, att2, b2):
    raise NotImplementedError("write your pallas kernel here")


# speedup vs baseline: 3.7493x; 3.7493x over previous
import functools

import jax
import jax.numpy as jnp
from jax import lax
from jax.experimental import pallas as pl
from jax.experimental.pallas import tpu as pltpu
from jax.experimental.pallas import tpu_sc as plsc

NN = 10000
EE = 320000
NPAD = 10240
ND8 = NPAD // 8
NS = 16
NW = NS
NHALF = NPAD // 2
ACCR = NHALF + 8
DENH = ND8 // 2
DENR = DENH + 8
CHK = 128
KCH = 162
EW = KCH * CHK
EPAD = NW * EW
NEG = 0.2
RPT = NHALF // NS
RPTD = DENH // NS
D = 128


def _matmul2_tc(x, Wl, Wr):
    n, d_in = x.shape
    d_out = Wl.shape[1]
    blk = 1024

    def body(x_ref, wl_ref, wr_ref, xl_ref, xr_ref):
        xb = x_ref[...]
        xl_ref[...] = jnp.dot(xb, wl_ref[...], preferred_element_type=jnp.float32)
        xr_ref[...] = jnp.dot(xb, wr_ref[...], preferred_element_type=jnp.float32)

    return pl.pallas_call(
        body,
        grid=(n // blk,),
        in_specs=[
            pl.BlockSpec((blk, d_in), lambda i: (i, 0)),
            pl.BlockSpec((d_in, d_out), lambda i: (0, 0)),
            pl.BlockSpec((d_in, d_out), lambda i: (0, 0)),
        ],
        out_specs=[
            pl.BlockSpec((blk, d_out), lambda i: (i, 0)),
            pl.BlockSpec((blk, d_out), lambda i: (i, 0)),
        ],
        out_shape=[jax.ShapeDtypeStruct((n, d_out), jnp.float32)] * 2,
    )(x, Wl, Wr)


@functools.partial(
    pl.kernel,
    out_type=[
        pltpu.HBM((NPAD, D), jnp.float32),
        pltpu.HBM((ND8, D), jnp.float32),
    ],
    mesh=plsc.VectorSubcoreMesh(core_axis_name="c", subcore_axis_name="s"),
    compiler_params=pltpu.CompilerParams(needs_layout_passes=False),
    scratch_types=[
        pltpu.VMEM((CHK,), jnp.int32),
        pltpu.VMEM((CHK,), jnp.int32),
        pltpu.VMEM((CHK + 16,), jnp.int32),
        pltpu.VMEM((CHK,), jnp.int32),
        pltpu.VMEM((CHK,), jnp.int32),
        pltpu.VMEM((CHK, D), jnp.float32),
        pltpu.VMEM((CHK, D), jnp.float32),
        pltpu.VMEM((CHK, D), jnp.float32),
        pltpu.VMEM((CHK, D), jnp.float32),
        pltpu.VMEM((CHK, D), jnp.float32),
        pltpu.VMEM((D,), jnp.float32),
        pltpu.VMEM_SHARED((ACCR, D), jnp.float32),
        pltpu.VMEM_SHARED((DENR, D), jnp.float32),
        pltpu.SemaphoreType.DMA,
        pltpu.SemaphoreType.DMA,
    ],
)
def _edge_kernel(xl_hbm, xr_hbm, src_hbm, dst_hbm, dstp_hbm, att_hbm,
                 out_hbm, den_hbm,
                 src_v, dst_v, dst_s, dstl_v, dst8_v, u_v, v_v, wu_v, wd_v,
                 z_v, att_v, acc_sh, den_sh, sem1, sem2):
    cid = lax.axis_index("c")
    sid = lax.axis_index("s")
    wid = sid
    pltpu.sync_copy(att_hbm, att_v)

    zeros16 = jnp.zeros((16,), jnp.float32)

    def zrow(r, carry):
        for j in range(D // 16):
            z_v[r, pl.ds(j * 16, 16)] = zeros16
            wd_v[r, pl.ds(j * 16, 16)] = zeros16
        return carry

    lax.fori_loop(0, CHK, zrow, 0)
    for t in range(RPT // CHK):
        pltpu.sync_copy(z_v, acc_sh.at[pl.ds(sid * RPT + t * CHK, CHK), :])
    pltpu.sync_copy(z_v.at[pl.ds(0, RPT % CHK), :],
                    acc_sh.at[pl.ds(sid * RPT + (RPT // CHK) * CHK,
                                    RPT % CHK), :])
    pltpu.sync_copy(z_v.at[pl.ds(0, RPTD), :],
                    den_sh.at[pl.ds(sid * RPTD, RPTD), :])

    @pl.when(sid == 0)
    def _zero_trash():
        pltpu.sync_copy(z_v.at[pl.ds(0, 8), :],
                        acc_sh.at[pl.ds(NHALF, 8), :])
        pltpu.sync_copy(z_v.at[pl.ds(0, 8), :],
                        den_sh.at[pl.ds(DENH, 8), :])

    plsc.subcore_barrier()

    lane = lax.broadcasted_iota(jnp.int32, (16,), 0)
    att16 = [att_v[pl.ds(j * 16, 16)] for j in range(D // 16)]

    def chunk(g, carry):
        base = wid * EW + g * CHK
        pltpu.sync_copy(src_hbm.at[pl.ds(base, CHK)], src_v)
        pltpu.sync_copy(dst_hbm.at[pl.ds(base, CHK)], dst_v)
        pltpu.sync_copy(dstp_hbm.at[pl.ds(base, CHK + 16)], dst_s)
        cp1 = pltpu.async_copy(xl_hbm.at[src_v], u_v, sem1)
        cp2 = pltpu.async_copy(xr_hbm.at[dst_v], v_v, sem2)
        lo = cid * NHALF
        lo8 = cid * DENH
        for eb in range(CHK // 16):
            d16 = dst_v[pl.ds(eb * 16, 16)]
            l16 = d16 - lo
            ok = jnp.logical_and(l16 >= 0, l16 < NHALF)
            dstl_v[pl.ds(eb * 16, 16)] = jnp.where(ok, l16, NHALF)
            l8 = lax.shift_right_logical(d16, 3) - lo8
            ok8 = jnp.logical_and(l8 >= 0, l8 < DENH)
            dst8_v[pl.ds(eb * 16, 16)] = jnp.where(ok8, l8, DENH)
        cp1.wait()
        cp2.wait()

        def edge(e, carry2):
            us = []
            acc = jnp.zeros((16,), jnp.float32)
            for j in range(D // 16):
                lu = u_v[e, pl.ds(j * 16, 16)]
                lv = v_v[e, pl.ds(j * 16, 16)]
                us.append(lu)
                zz = lu + lv
                zz = jnp.maximum(zz, NEG * zz)
                acc = acc + att16[j] * zz
            logit = jnp.sum(acc, axis=0)
            wvec = jnp.exp(jnp.full((16,), 1.0, jnp.float32) * logit)
            for j in range(D // 16):
                wu_v[e, pl.ds(j * 16, 16)] = us[j] * wvec
            d16 = dst_s[pl.ds(e, 16)]
            dloc = d16[0]
            col = lax.mul(lax.rem(dloc, 8), 16)
            for j in range(D // 16):
                wd_v[e, pl.ds(j * 16, 16)] = zeros16
            wd_v[e, pl.ds(col, 16)] = jnp.where(lane == 0, wvec, zeros16)
            return carry2

        lax.fori_loop(0, CHK, edge, 0)
        pltpu.sync_copy(wu_v, acc_sh.at[dstl_v], add=True)
        pltpu.sync_copy(wd_v, den_sh.at[dst8_v], add=True)
        return carry

    lax.fori_loop(0, KCH, chunk, 0)
    plsc.subcore_barrier()
    pltpu.sync_copy(
        acc_sh.at[pl.ds(sid * RPT, RPT), :],
        out_hbm.at[pl.ds(cid * NHALF + sid * RPT, RPT), :])
    pltpu.sync_copy(
        den_sh.at[pl.ds(sid * RPTD, RPTD), :],
        den_hbm.at[pl.ds(cid * DENH + sid * RPTD, RPTD), :])


def _finish(P, dn, b, g, be, flag):
    def body(p_ref, d_ref, b_ref, g_ref, be_ref, f_ref, o_ref):
        num = p_ref[:NN, :]
        den = d_ref[:NN, :]
        h = num / (den + 1e-16) + b_ref[...]
        mean = jnp.mean(h, axis=0)
        var = jnp.mean((h - mean) ** 2, axis=0)
        hn = (h - mean) / jnp.sqrt(var + 1e-5) * g_ref[...] + be_ref[...]
        bn = jnp.maximum(hn, 0.0)
        h64 = h[:, :64]
        m = jnp.max(h64, axis=1, keepdims=True)
        lse = m + jnp.log(jnp.sum(jnp.exp(h64 - m), axis=1, keepdims=True))
        ls = jnp.concatenate(
            [h64 - lse, jnp.zeros((NN, 64), jnp.float32)], axis=1)
        o_ref[...] = jnp.where(f_ref[0, 0] > 0.5, ls, bn)

    return pl.pallas_call(
        body,
        out_shape=jax.ShapeDtypeStruct((NN, D), jnp.float32),
    )(P, dn, b.reshape(1, D), g.reshape(1, D), be.reshape(1, D), flag)


def _den_cols(Pd):
    return Pd.reshape(ND8, 8, 16)[..., 0].reshape(NPAD, 1)


def kernel(x, edge_index, Wl0, Wr0, att0, b0, g0, be0,
           Wl1, Wr1, att1, b1, g1, be1, Wl2, Wr2, att2, b2):
    loops = jnp.arange(NN, dtype=jnp.int32)
    pad = jnp.full((EPAD - EE - NN,), NN, jnp.int32)
    src = jnp.concatenate([edge_index[0].astype(jnp.int32), loops, pad])
    dst = jnp.concatenate([edge_index[1].astype(jnp.int32), loops, pad])
    dstp = jnp.concatenate([dst, jnp.full((16,), NN, jnp.int32)])

    Wl2p = jnp.zeros((D, D), jnp.float32).at[:, :64].set(Wl2)
    Wr2p = jnp.zeros((D, D), jnp.float32).at[:, :64].set(Wr2)
    att2p = jnp.zeros((D,), jnp.float32).at[:64].set(att2)
    b2p = jnp.zeros((D,), jnp.float32).at[:64].set(b2)

    Wls = jnp.stack([Wl0, Wl1, Wl2p])
    Wrs = jnp.stack([Wr0, Wr1, Wr2p])
    atts = jnp.stack([att0, att1, att2p])
    bs = jnp.stack([b0, b1, b2p])
    gs = jnp.stack([g0, g1, jnp.ones((D,), jnp.float32)])
    bes = jnp.stack([be0, be1, jnp.zeros((D,), jnp.float32)])
    flags = jnp.array([0.0, 0.0, 1.0], jnp.float32).reshape(3, 1, 1)

    nl = jnp.int32(3) + (src[0] - src[0])

    def cond(c):
        i, _ = c
        return i < nl

    def body(c):
        i, h = c
        Wl = lax.dynamic_index_in_dim(Wls, i, 0, keepdims=False)
        Wr = lax.dynamic_index_in_dim(Wrs, i, 0, keepdims=False)
        att = lax.dynamic_index_in_dim(atts, i, 0, keepdims=False)
        b = lax.dynamic_index_in_dim(bs, i, 0, keepdims=False)
        g = lax.dynamic_index_in_dim(gs, i, 0, keepdims=False)
        be = lax.dynamic_index_in_dim(bes, i, 0, keepdims=False)
        fl = lax.dynamic_index_in_dim(flags, i, 0, keepdims=False)
        hp = jnp.zeros((NPAD, D), jnp.float32).at[:NN].set(h)
        xl, xr = _matmul2_tc(hp, Wl, Wr)
        P, Pd = _edge_kernel(xl, xr, src, dst, dstp, att)
        h2 = _finish(P, _den_cols(Pd), b, g, be, fl)
        return (i + jnp.int32(1), h2)

    _, h_final = lax.while_loop(cond, body, (jnp.int32(0), x))
    return h_final[:, :64]


# speedup vs baseline: 3.7788x; 1.0079x over previous
import functools

import jax
import jax.numpy as jnp
from jax import lax
from jax.experimental import pallas as pl
from jax.experimental.pallas import tpu as pltpu
from jax.experimental.pallas import tpu_sc as plsc

NN = 10000
EE = 320000
NPAD = 10240
ND8 = NPAD // 8
NS = 16
NW = NS
NHALF = NPAD // 2
ACCR = NHALF + 8
DENH = ND8 // 2
DENR = DENH + 8
CHK = 128
KCH = 162
EW = KCH * CHK
EPAD = NW * EW
NEG = 0.2
RPT = NHALF // NS
RPTD = DENH // NS
D = 128


def _matmul2_tc(x, Wl, Wr):
    n, d_in = x.shape
    d_out = Wl.shape[1]
    blk = 1024

    def body(x_ref, wl_ref, wr_ref, xl_ref, xr_ref):
        xb = x_ref[...]
        xl_ref[...] = jnp.dot(xb, wl_ref[...], preferred_element_type=jnp.float32)
        xr_ref[...] = jnp.dot(xb, wr_ref[...], preferred_element_type=jnp.float32)

    return pl.pallas_call(
        body,
        grid=(n // blk,),
        in_specs=[
            pl.BlockSpec((blk, d_in), lambda i: (i, 0)),
            pl.BlockSpec((d_in, d_out), lambda i: (0, 0)),
            pl.BlockSpec((d_in, d_out), lambda i: (0, 0)),
        ],
        out_specs=[
            pl.BlockSpec((blk, d_out), lambda i: (i, 0)),
            pl.BlockSpec((blk, d_out), lambda i: (i, 0)),
        ],
        out_shape=[jax.ShapeDtypeStruct((n, d_out), jnp.float32)] * 2,
    )(x, Wl, Wr)


@functools.partial(
    pl.kernel,
    out_type=[
        pltpu.HBM((NPAD, D), jnp.float32),
        pltpu.HBM((ND8, D), jnp.float32),
    ],
    mesh=plsc.VectorSubcoreMesh(core_axis_name="c", subcore_axis_name="s"),
    compiler_params=pltpu.CompilerParams(needs_layout_passes=False),
    scratch_types=[
        pltpu.VMEM((CHK,), jnp.int32),
        pltpu.VMEM((CHK,), jnp.int32),
        pltpu.VMEM((CHK + 16,), jnp.int32),
        pltpu.VMEM((CHK,), jnp.int32),
        pltpu.VMEM((CHK,), jnp.int32),
        pltpu.VMEM((CHK, D), jnp.float32),
        pltpu.VMEM((CHK, D), jnp.float32),
        pltpu.VMEM((CHK, D), jnp.float32),
        pltpu.VMEM((CHK, D), jnp.float32),
        pltpu.VMEM((CHK, D), jnp.float32),
        pltpu.VMEM((D,), jnp.float32),
        pltpu.VMEM_SHARED((ACCR, D), jnp.float32),
        pltpu.VMEM_SHARED((DENR, D), jnp.float32),
        pltpu.SemaphoreType.DMA,
        pltpu.SemaphoreType.DMA,
    ],
)
def _edge_kernel(xl_hbm, xr_hbm, src_hbm, dst_hbm, dstp_hbm, att_hbm,
                 out_hbm, den_hbm,
                 src_v, dst_v, dst_s, dstl_v, dst8_v, u_v, v_v, wu_v, wd_v,
                 z_v, att_v, acc_sh, den_sh, sem1, sem2):
    cid = lax.axis_index("c")
    sid = lax.axis_index("s")
    wid = sid
    pltpu.sync_copy(att_hbm, att_v)

    zeros16 = jnp.zeros((16,), jnp.float32)

    def zrow(r, carry):
        for j in range(D // 16):
            z_v[r, pl.ds(j * 16, 16)] = zeros16
            wd_v[r, pl.ds(j * 16, 16)] = zeros16
        return carry

    lax.fori_loop(0, CHK, zrow, 0)
    for t in range(RPT // CHK):
        pltpu.sync_copy(z_v, acc_sh.at[pl.ds(sid * RPT + t * CHK, CHK), :])
    pltpu.sync_copy(z_v.at[pl.ds(0, RPT % CHK), :],
                    acc_sh.at[pl.ds(sid * RPT + (RPT // CHK) * CHK,
                                    RPT % CHK), :])
    pltpu.sync_copy(z_v.at[pl.ds(0, RPTD), :],
                    den_sh.at[pl.ds(sid * RPTD, RPTD), :])

    @pl.when(sid == 0)
    def _zero_trash():
        pltpu.sync_copy(z_v.at[pl.ds(0, 8), :],
                        acc_sh.at[pl.ds(NHALF, 8), :])
        pltpu.sync_copy(z_v.at[pl.ds(0, 8), :],
                        den_sh.at[pl.ds(DENH, 8), :])

    plsc.subcore_barrier()

    lane = lax.broadcasted_iota(jnp.int32, (16,), 0)
    att16 = [att_v[pl.ds(j * 16, 16)] for j in range(D // 16)]

    def chunk(g, carry):
        base = wid * EW + g * CHK
        pltpu.sync_copy(src_hbm.at[pl.ds(base, CHK)], src_v)
        pltpu.sync_copy(dst_hbm.at[pl.ds(base, CHK)], dst_v)
        pltpu.sync_copy(dstp_hbm.at[pl.ds(base, CHK + 16)], dst_s)
        cp1 = pltpu.async_copy(xl_hbm.at[src_v], u_v, sem1)
        cp2 = pltpu.async_copy(xr_hbm.at[dst_v], v_v, sem2)
        lo = cid * NHALF
        lo8 = cid * DENH
        for eb in range(CHK // 16):
            d16 = dst_v[pl.ds(eb * 16, 16)]
            l16 = d16 - lo
            ok = jnp.logical_and(l16 >= 0, l16 < NHALF)
            dstl_v[pl.ds(eb * 16, 16)] = jnp.where(ok, l16, NHALF)
            l8 = lax.shift_right_logical(d16, 3) - lo8
            ok8 = jnp.logical_and(l8 >= 0, l8 < DENH)
            dst8_v[pl.ds(eb * 16, 16)] = jnp.where(ok8, l8, DENH)
        cp1.wait()
        cp2.wait()

        def edge(e, carry2):
            us = []
            acc = jnp.zeros((16,), jnp.float32)
            for j in range(D // 16):
                lu = u_v[e, pl.ds(j * 16, 16)]
                lv = v_v[e, pl.ds(j * 16, 16)]
                us.append(lu)
                zz = lu + lv
                zz = jnp.maximum(zz, NEG * zz)
                acc = acc + att16[j] * zz
            logit = jnp.sum(acc, axis=0)
            wvec = jnp.exp(jnp.full((16,), 1.0, jnp.float32) * logit)
            for j in range(D // 16):
                wu_v[e, pl.ds(j * 16, 16)] = us[j] * wvec
            d16 = dst_s[pl.ds(e, 16)]
            dloc = d16[0]
            col = lax.mul(lax.rem(dloc, 8), 16)
            for j in range(D // 16):
                wd_v[e, pl.ds(j * 16, 16)] = zeros16
            wd_v[e, pl.ds(col, 16)] = jnp.where(lane == 0, wvec, zeros16)
            return carry2

        lax.fori_loop(0, CHK, edge, 0, unroll=4)
        pltpu.sync_copy(wu_v, acc_sh.at[dstl_v], add=True)
        pltpu.sync_copy(wd_v, den_sh.at[dst8_v], add=True)
        return carry

    lax.fori_loop(0, KCH, chunk, 0)
    plsc.subcore_barrier()
    pltpu.sync_copy(
        acc_sh.at[pl.ds(sid * RPT, RPT), :],
        out_hbm.at[pl.ds(cid * NHALF + sid * RPT, RPT), :])
    pltpu.sync_copy(
        den_sh.at[pl.ds(sid * RPTD, RPTD), :],
        den_hbm.at[pl.ds(cid * DENH + sid * RPTD, RPTD), :])


def _finish(P, dn, b, g, be, flag):
    def body(p_ref, d_ref, b_ref, g_ref, be_ref, f_ref, o_ref):
        num = p_ref[:NN, :]
        den = d_ref[:NN, :]
        h = num / (den + 1e-16) + b_ref[...]
        mean = jnp.mean(h, axis=0)
        var = jnp.mean((h - mean) ** 2, axis=0)
        hn = (h - mean) / jnp.sqrt(var + 1e-5) * g_ref[...] + be_ref[...]
        bn = jnp.maximum(hn, 0.0)
        h64 = h[:, :64]
        m = jnp.max(h64, axis=1, keepdims=True)
        lse = m + jnp.log(jnp.sum(jnp.exp(h64 - m), axis=1, keepdims=True))
        ls = jnp.concatenate(
            [h64 - lse, jnp.zeros((NN, 64), jnp.float32)], axis=1)
        o_ref[...] = jnp.where(f_ref[0, 0] > 0.5, ls, bn)

    return pl.pallas_call(
        body,
        out_shape=jax.ShapeDtypeStruct((NN, D), jnp.float32),
    )(P, dn, b.reshape(1, D), g.reshape(1, D), be.reshape(1, D), flag)


def _den_cols(Pd):
    return Pd.reshape(ND8, 8, 16)[..., 0].reshape(NPAD, 1)


def kernel(x, edge_index, Wl0, Wr0, att0, b0, g0, be0,
           Wl1, Wr1, att1, b1, g1, be1, Wl2, Wr2, att2, b2):
    loops = jnp.arange(NN, dtype=jnp.int32)
    pad = jnp.full((EPAD - EE - NN,), NN, jnp.int32)
    src = jnp.concatenate([edge_index[0].astype(jnp.int32), loops, pad])
    dst = jnp.concatenate([edge_index[1].astype(jnp.int32), loops, pad])
    dstp = jnp.concatenate([dst, jnp.full((16,), NN, jnp.int32)])

    Wl2p = jnp.zeros((D, D), jnp.float32).at[:, :64].set(Wl2)
    Wr2p = jnp.zeros((D, D), jnp.float32).at[:, :64].set(Wr2)
    att2p = jnp.zeros((D,), jnp.float32).at[:64].set(att2)
    b2p = jnp.zeros((D,), jnp.float32).at[:64].set(b2)

    Wls = jnp.stack([Wl0, Wl1, Wl2p])
    Wrs = jnp.stack([Wr0, Wr1, Wr2p])
    atts = jnp.stack([att0, att1, att2p])
    bs = jnp.stack([b0, b1, b2p])
    gs = jnp.stack([g0, g1, jnp.ones((D,), jnp.float32)])
    bes = jnp.stack([be0, be1, jnp.zeros((D,), jnp.float32)])
    flags = jnp.array([0.0, 0.0, 1.0], jnp.float32).reshape(3, 1, 1)

    nl = jnp.int32(3) + (src[0] - src[0])

    def cond(c):
        i, _ = c
        return i < nl

    def body(c):
        i, h = c
        Wl = lax.dynamic_index_in_dim(Wls, i, 0, keepdims=False)
        Wr = lax.dynamic_index_in_dim(Wrs, i, 0, keepdims=False)
        att = lax.dynamic_index_in_dim(atts, i, 0, keepdims=False)
        b = lax.dynamic_index_in_dim(bs, i, 0, keepdims=False)
        g = lax.dynamic_index_in_dim(gs, i, 0, keepdims=False)
        be = lax.dynamic_index_in_dim(bes, i, 0, keepdims=False)
        fl = lax.dynamic_index_in_dim(flags, i, 0, keepdims=False)
        hp = jnp.zeros((NPAD, D), jnp.float32).at[:NN].set(h)
        xl, xr = _matmul2_tc(hp, Wl, Wr)
        P, Pd = _edge_kernel(xl, xr, src, dst, dstp, att)
        h2 = _finish(P, _den_cols(Pd), b, g, be, fl)
        return (i + jnp.int32(1), h2)

    _, h_final = lax.while_loop(cond, body, (jnp.int32(0), x))
    return h_final[:, :64]


# speedup vs baseline: 3.7876x; 1.0023x over previous
import functools

import jax
import jax.numpy as jnp
from jax import lax
from jax.experimental import pallas as pl
from jax.experimental.pallas import tpu as pltpu
from jax.experimental.pallas import tpu_sc as plsc

NN = 10000
EE = 320000
NPAD = 10240
ND8 = NPAD // 8
NS = 16
NW = NS
NHALF = NPAD // 2
ACCR = NHALF + 8
DENH = ND8 // 2
DENR = DENH + 8
CHK = 128
KCH = 162
EW = KCH * CHK
EPAD = NW * EW
NEG = 0.2
RPT = NHALF // NS
RPTD = DENH // NS
D = 128


def _matmul2_tc(x, Wl, Wr):
    n, d_in = x.shape
    d_out = Wl.shape[1]
    blk = 1024

    def body(x_ref, wl_ref, wr_ref, xl_ref, xr_ref):
        xb = x_ref[...]
        xl_ref[...] = jnp.dot(xb, wl_ref[...], preferred_element_type=jnp.float32)
        xr_ref[...] = jnp.dot(xb, wr_ref[...], preferred_element_type=jnp.float32)

    return pl.pallas_call(
        body,
        grid=(n // blk,),
        in_specs=[
            pl.BlockSpec((blk, d_in), lambda i: (i, 0)),
            pl.BlockSpec((d_in, d_out), lambda i: (0, 0)),
            pl.BlockSpec((d_in, d_out), lambda i: (0, 0)),
        ],
        out_specs=[
            pl.BlockSpec((blk, d_out), lambda i: (i, 0)),
            pl.BlockSpec((blk, d_out), lambda i: (i, 0)),
        ],
        out_shape=[jax.ShapeDtypeStruct((n, d_out), jnp.float32)] * 2,
    )(x, Wl, Wr)


@functools.partial(
    pl.kernel,
    out_type=[
        pltpu.HBM((NPAD, D), jnp.float32),
        pltpu.HBM((ND8, D), jnp.float32),
    ],
    mesh=plsc.VectorSubcoreMesh(core_axis_name="c", subcore_axis_name="s"),
    compiler_params=pltpu.CompilerParams(needs_layout_passes=False),
    scratch_types=[
        pltpu.VMEM((CHK,), jnp.int32),
        pltpu.VMEM((CHK,), jnp.int32),
        pltpu.VMEM((CHK + 16,), jnp.int32),
        pltpu.VMEM((CHK,), jnp.int32),
        pltpu.VMEM((CHK,), jnp.int32),
        pltpu.VMEM((CHK, D), jnp.float32),
        pltpu.VMEM((CHK, D), jnp.float32),
        pltpu.VMEM((CHK, D), jnp.float32),
        pltpu.VMEM((CHK, D), jnp.float32),
        pltpu.VMEM((CHK, D), jnp.float32),
        pltpu.VMEM((D,), jnp.float32),
        pltpu.VMEM_SHARED((ACCR, D), jnp.float32),
        pltpu.VMEM_SHARED((DENR, D), jnp.float32),
        pltpu.SemaphoreType.DMA,
        pltpu.SemaphoreType.DMA,
        pltpu.SemaphoreType.DMA,
        pltpu.SemaphoreType.DMA,
    ],
)
def _edge_kernel(xl_hbm, xr_hbm, src_hbm, dst_hbm, dstp_hbm, att_hbm,
                 out_hbm, den_hbm,
                 src_v, dst_v, dst_s, dstl_v, dst8_v, u_v, v_v, wu_v, wd_v,
                 z_v, att_v, acc_sh, den_sh, sem1, sem2, sem3, sem4):
    cid = lax.axis_index("c")
    sid = lax.axis_index("s")
    wid = sid
    pltpu.sync_copy(att_hbm, att_v)

    zeros16 = jnp.zeros((16,), jnp.float32)

    def zrow(r, carry):
        for j in range(D // 16):
            z_v[r, pl.ds(j * 16, 16)] = zeros16
            wd_v[r, pl.ds(j * 16, 16)] = zeros16
        return carry

    lax.fori_loop(0, CHK, zrow, 0)
    for t in range(RPT // CHK):
        pltpu.sync_copy(z_v, acc_sh.at[pl.ds(sid * RPT + t * CHK, CHK), :])
    pltpu.sync_copy(z_v.at[pl.ds(0, RPT % CHK), :],
                    acc_sh.at[pl.ds(sid * RPT + (RPT // CHK) * CHK,
                                    RPT % CHK), :])
    pltpu.sync_copy(z_v.at[pl.ds(0, RPTD), :],
                    den_sh.at[pl.ds(sid * RPTD, RPTD), :])

    @pl.when(sid == 0)
    def _zero_trash():
        pltpu.sync_copy(z_v.at[pl.ds(0, 8), :],
                        acc_sh.at[pl.ds(NHALF, 8), :])
        pltpu.sync_copy(z_v.at[pl.ds(0, 8), :],
                        den_sh.at[pl.ds(DENH, 8), :])

    plsc.subcore_barrier()

    lane = lax.broadcasted_iota(jnp.int32, (16,), 0)
    att16 = [att_v[pl.ds(j * 16, 16)] for j in range(D // 16)]

    def chunk(g, carry):
        base = wid * EW + g * CHK
        pltpu.sync_copy(src_hbm.at[pl.ds(base, CHK)], src_v)
        pltpu.sync_copy(dst_hbm.at[pl.ds(base, CHK)], dst_v)
        pltpu.sync_copy(dstp_hbm.at[pl.ds(base, CHK + 16)], dst_s)
        cp1 = pltpu.async_copy(xl_hbm.at[src_v], u_v, sem1)
        cp2 = pltpu.async_copy(xr_hbm.at[dst_v], v_v, sem2)
        lo = cid * NHALF
        lo8 = cid * DENH
        for eb in range(CHK // 16):
            d16 = dst_v[pl.ds(eb * 16, 16)]
            l16 = d16 - lo
            ok = jnp.logical_and(l16 >= 0, l16 < NHALF)
            dstl_v[pl.ds(eb * 16, 16)] = jnp.where(ok, l16, NHALF)
            l8 = lax.shift_right_logical(d16, 3) - lo8
            ok8 = jnp.logical_and(l8 >= 0, l8 < DENH)
            dst8_v[pl.ds(eb * 16, 16)] = jnp.where(ok8, l8, DENH)
        cp1.wait()
        cp2.wait()

        def edge(e, carry2):
            us = []
            acc = jnp.zeros((16,), jnp.float32)
            for j in range(D // 16):
                lu = u_v[e, pl.ds(j * 16, 16)]
                lv = v_v[e, pl.ds(j * 16, 16)]
                us.append(lu)
                zz = lu + lv
                zz = jnp.maximum(zz, NEG * zz)
                acc = acc + att16[j] * zz
            logit = jnp.sum(acc, axis=0)
            wvec = jnp.exp(jnp.full((16,), 1.0, jnp.float32) * logit)
            for j in range(D // 16):
                wu_v[e, pl.ds(j * 16, 16)] = us[j] * wvec
            d16 = dst_s[pl.ds(e, 16)]
            dloc = d16[0]
            col = lax.mul(lax.rem(dloc, 8), 16)
            for j in range(D // 16):
                wd_v[e, pl.ds(j * 16, 16)] = zeros16
            wd_v[e, pl.ds(col, 16)] = jnp.where(lane == 0, wvec, zeros16)
            return carry2

        lax.fori_loop(0, CHK, edge, 0, unroll=4)
        cp3 = pltpu.async_copy(wu_v, acc_sh.at[dstl_v], sem3, add=True)
        cp4 = pltpu.async_copy(wd_v, den_sh.at[dst8_v], sem4, add=True)
        cp3.wait()
        cp4.wait()
        return carry

    lax.fori_loop(0, KCH, chunk, 0)
    plsc.subcore_barrier()
    pltpu.sync_copy(
        acc_sh.at[pl.ds(sid * RPT, RPT), :],
        out_hbm.at[pl.ds(cid * NHALF + sid * RPT, RPT), :])
    pltpu.sync_copy(
        den_sh.at[pl.ds(sid * RPTD, RPTD), :],
        den_hbm.at[pl.ds(cid * DENH + sid * RPTD, RPTD), :])


def _finish(P, dn, b, g, be, flag):
    def body(p_ref, d_ref, b_ref, g_ref, be_ref, f_ref, o_ref):
        num = p_ref[:NN, :]
        den = d_ref[:NN, :]
        h = num / (den + 1e-16) + b_ref[...]
        mean = jnp.mean(h, axis=0)
        var = jnp.mean((h - mean) ** 2, axis=0)
        hn = (h - mean) / jnp.sqrt(var + 1e-5) * g_ref[...] + be_ref[...]
        bn = jnp.maximum(hn, 0.0)
        h64 = h[:, :64]
        m = jnp.max(h64, axis=1, keepdims=True)
        lse = m + jnp.log(jnp.sum(jnp.exp(h64 - m), axis=1, keepdims=True))
        ls = jnp.concatenate(
            [h64 - lse, jnp.zeros((NN, 64), jnp.float32)], axis=1)
        o_ref[...] = jnp.where(f_ref[0, 0] > 0.5, ls, bn)

    return pl.pallas_call(
        body,
        out_shape=jax.ShapeDtypeStruct((NN, D), jnp.float32),
    )(P, dn, b.reshape(1, D), g.reshape(1, D), be.reshape(1, D), flag)


def _den_cols(Pd):
    return Pd.reshape(ND8, 8, 16)[..., 0].reshape(NPAD, 1)


def kernel(x, edge_index, Wl0, Wr0, att0, b0, g0, be0,
           Wl1, Wr1, att1, b1, g1, be1, Wl2, Wr2, att2, b2):
    loops = jnp.arange(NN, dtype=jnp.int32)
    pad = jnp.full((EPAD - EE - NN,), NN, jnp.int32)
    src = jnp.concatenate([edge_index[0].astype(jnp.int32), loops, pad])
    dst = jnp.concatenate([edge_index[1].astype(jnp.int32), loops, pad])
    dstp = jnp.concatenate([dst, jnp.full((16,), NN, jnp.int32)])

    Wl2p = jnp.zeros((D, D), jnp.float32).at[:, :64].set(Wl2)
    Wr2p = jnp.zeros((D, D), jnp.float32).at[:, :64].set(Wr2)
    att2p = jnp.zeros((D,), jnp.float32).at[:64].set(att2)
    b2p = jnp.zeros((D,), jnp.float32).at[:64].set(b2)

    Wls = jnp.stack([Wl0, Wl1, Wl2p])
    Wrs = jnp.stack([Wr0, Wr1, Wr2p])
    atts = jnp.stack([att0, att1, att2p])
    bs = jnp.stack([b0, b1, b2p])
    gs = jnp.stack([g0, g1, jnp.ones((D,), jnp.float32)])
    bes = jnp.stack([be0, be1, jnp.zeros((D,), jnp.float32)])
    flags = jnp.array([0.0, 0.0, 1.0], jnp.float32).reshape(3, 1, 1)

    nl = jnp.int32(3) + (src[0] - src[0])

    def cond(c):
        i, _ = c
        return i < nl

    def body(c):
        i, h = c
        Wl = lax.dynamic_index_in_dim(Wls, i, 0, keepdims=False)
        Wr = lax.dynamic_index_in_dim(Wrs, i, 0, keepdims=False)
        att = lax.dynamic_index_in_dim(atts, i, 0, keepdims=False)
        b = lax.dynamic_index_in_dim(bs, i, 0, keepdims=False)
        g = lax.dynamic_index_in_dim(gs, i, 0, keepdims=False)
        be = lax.dynamic_index_in_dim(bes, i, 0, keepdims=False)
        fl = lax.dynamic_index_in_dim(flags, i, 0, keepdims=False)
        hp = jnp.zeros((NPAD, D), jnp.float32).at[:NN].set(h)
        xl, xr = _matmul2_tc(hp, Wl, Wr)
        P, Pd = _edge_kernel(xl, xr, src, dst, dstp, att)
        h2 = _finish(P, _den_cols(Pd), b, g, be, fl)
        return (i + jnp.int32(1), h2)

    _, h_final = lax.while_loop(cond, body, (jnp.int32(0), x))
    return h_final[:, :64]
